# baseline (speedup 1.0000x reference)
import functools

import jax
import jax.numpy as jnp
from jax import lax
from jax.experimental import pallas as pl
from jax.experimental.pallas import tpu as pltpu
from jax.experimental.pallas import tpu_sc as plsc

_U = 25000
_N = 50000
_E = 800000
_D = 64
_H = 128
_TEMP = 0.2
_REG_L = 1e-4
_SSL_L = 0.1
_B = 4096

_NC = 2
_NS = 16
_CHUNK = 128
_CHT = 392
_EPT = _CHT * _CHUNK
_EPAD = _EPT * _NS
_DUMP = _U
_ACC_ROWS = 25088


def _spmm_body(packed_ref, emb_ref, zeros_ref, out_ref,
               edg0, edg1, dl0, dl1, rows0, rows1,
               acc_sh, smi0, smi1, smg0, smg1):
    c = lax.axis_index("c")
    s = lax.axis_index("s")
    lo = c * _U

    zb = s * (_ACC_ROWS // _NS)
    pltpu.sync_copy(zeros_ref.at[pl.ds(zb, _ACC_ROWS // _NS)],
                    acc_sh.at[pl.ds(zb, _ACC_ROWS // _NS)])
    plsc.subcore_barrier()

    ebase = s * _EPT
    maxoff = _EPAD - _CHUNK

    def _off(k):
        return jnp.minimum(ebase + k * _CHUNK, maxoff)

    def start_edges(k, edg, sem):
        pltpu.async_copy(packed_ref.at[:, pl.ds(_off(k), _CHUNK)], edg, sem)

    def wait_idx(edg, sem):
        pltpu.make_async_copy(packed_ref.at[:, pl.ds(0, _CHUNK)], edg,
                              sem).wait()

    def start_gather(edg, rows, sem):
        pltpu.async_copy(emb_ref.at[edg.at[1]], rows, sem)

    def wait_gather(edg, rows, sem):
        pltpu.make_async_copy(emb_ref.at[edg.at[1]], rows, sem).wait()

    def compute_dl(edg, dl):
        for j in range(_CHUNK // 16):
            d16 = edg[0, pl.ds(j * 16, 16)]
            inr = (d16 >= lo) & (d16 < lo + _U)
            dl[pl.ds(j * 16, 16)] = jnp.where(inr, d16 - lo, _DUMP)

    def scale(v16s, rows):
        for j in range(_CHUNK // 16):
            v16 = v16s[j]
            for e in range(16):
                vs = jnp.full((16,), v16[e])
                row = j * 16 + e
                for q in range(_D // 16):
                    rows[row, pl.ds(q * 16, 16)] = (
                        rows[row, pl.ds(q * 16, 16)] * vs)

    def process(k_next, edg, dl, rows, smi, smg):
        wait_gather(edg, rows, smg)
        compute_dl(edg, dl)
        v16s = [lax.bitcast_convert_type(edg[2, pl.ds(j * 16, 16)],
                                         jnp.float32)
                for j in range(_CHUNK // 16)]
        start_edges(k_next, edg, smi)
        scale(v16s, rows)
        pltpu.sync_copy(rows, acc_sh.at[dl], add=True)
        wait_idx(edg, smi)
        start_gather(edg, rows, smg)

    start_edges(0, edg0, smi0)
    start_edges(1, edg1, smi1)
    wait_idx(edg0, smi0)
    start_gather(edg0, rows0, smg0)
    wait_idx(edg1, smi1)
    start_gather(edg1, rows1, smg1)

    def pair(i, carry):
        k = 2 * i
        process(k + 2, edg0, dl0, rows0, smi0, smg0)
        process(k + 3, edg1, dl1, rows1, smi1, smg1)
        return carry

    lax.fori_loop(0, _CHT // 2, pair, 0)

    wait_gather(edg0, rows0, smg0)
    wait_gather(edg1, rows1, smg1)
    plsc.subcore_barrier()

    @pl.when(s < 15)
    def _():
        base = s * 1568
        pltpu.sync_copy(acc_sh.at[pl.ds(base, 1568)],
                        out_ref.at[pl.ds(lo + base, 1568)])

    @pl.when(s == 15)
    def _():
        pltpu.sync_copy(acc_sh.at[pl.ds(15 * 1568, 1480)],
                        out_ref.at[pl.ds(lo + 15 * 1568, 1480)])


@functools.cache
def _spmm():
    return pl.kernel(
        _spmm_body,
        mesh=plsc.VectorSubcoreMesh(core_axis_name="c", subcore_axis_name="s", num_cores=_NC, num_subcores=_NS),
        out_type=jax.ShapeDtypeStruct((_N, _D), jnp.float32),
        compiler_params=pltpu.CompilerParams(use_tc_tiling_on_sc=False),
        scratch_types=[
            pltpu.VMEM((3, _CHUNK), jnp.int32),
            pltpu.VMEM((3, _CHUNK), jnp.int32),
            pltpu.VMEM((_CHUNK,), jnp.int32),
            pltpu.VMEM((_CHUNK,), jnp.int32),
            pltpu.VMEM((_CHUNK, _D), jnp.float32),
            pltpu.VMEM((_CHUNK, _D), jnp.float32),
            pltpu.VMEM_SHARED((_ACC_ROWS, _D), jnp.float32),
            pltpu.SemaphoreType.DMA,
            pltpu.SemaphoreType.DMA,
            pltpu.SemaphoreType.DMA,
            pltpu.SemaphoreType.DMA,
        ],
    )


_SMAP = (0, 0, 0, 1, 1, 1, 2, 3, 2, 3, 4, 5, 4, 5)


def _gather_body(idx_ref, s0, s1, s2, s3, s4, s5, out_ref, idx_v, rows_v, sem):
    w = lax.axis_index("s") * _NC + lax.axis_index("c")
    base = w * 128
    srcs = (s0, s1, s2, s3, s4, s5)
    for t in range(14):
        pltpu.sync_copy(idx_ref.at[t, pl.ds(base, 128)], idx_v)
        pltpu.async_copy(srcs[_SMAP[t]].at[idx_v], rows_v, sem).wait()
        pltpu.sync_copy(rows_v, out_ref.at[t, pl.ds(base, 128)])


@functools.cache
def _gather():
    return pl.kernel(
        _gather_body,
        mesh=plsc.VectorSubcoreMesh(core_axis_name="c", subcore_axis_name="s", num_cores=_NC, num_subcores=_NS),
        out_type=jax.ShapeDtypeStruct((14, _B, _D), jnp.float32),
        compiler_params=pltpu.CompilerParams(use_tc_tiling_on_sc=False),
        scratch_types=[
            pltpu.VMEM((128,), jnp.int32),
            pltpu.VMEM((128, _D), jnp.float32),
            pltpu.SemaphoreType.DMA,
        ],
    )


_RB = 1000


def _hyper_body(e_ref, p_ref, o_ref):
    o_ref[0] = jnp.dot(e_ref[...], p_ref[0], preferred_element_type=jnp.float32)


_hyper = pl.pallas_call(
    _hyper_body,
    grid=(2, _U // _RB),
    in_specs=[pl.BlockSpec((_RB, _D), lambda t, i: (t * (_U // _RB) + i, 0)),
              pl.BlockSpec((1, _D, _H), lambda t, i: (t, 0, 0))],
    out_specs=pl.BlockSpec((1, _RB, _H), lambda t, i: (t, i, 0)),
    out_shape=jax.ShapeDtypeStruct((2, _U, _H), jnp.float32),
)


def _latent_body(h_ref, e_ref, o_ref):
    i = pl.program_id(1)

    @pl.when(i == 0)
    def _():
        o_ref[...] = jnp.zeros_like(o_ref)

    a = h_ref[0]
    b = e_ref[...]
    o_ref[...] += lax.dot_general(a, b, (((0,), (0,)), ((), ())),
                                  preferred_element_type=jnp.float32)[None]


_latent = pl.pallas_call(
    _latent_body,
    grid=(2, _U // _RB),
    in_specs=[pl.BlockSpec((1, _RB, _H), lambda t, i: (t, i, 0)),
              pl.BlockSpec((_RB, _D), lambda t, i: (t * (_U // _RB) + i, 0))],
    out_specs=pl.BlockSpec((1, _H, _D), lambda t, i: (t, 0, 0)),
    out_shape=jax.ShapeDtypeStruct((2, _H, _D), jnp.float32),
)


def _prop_body(h_ref, l_ref, g_ref, a_ref, emb_o, h_o, acc_o):
    hblk = jnp.dot(h_ref[0], l_ref[0], preferred_element_type=jnp.float32)
    e = g_ref[...] + hblk
    emb_o[...] = e
    h_o[...] = hblk
    acc_o[...] = a_ref[...] + e


_prop = pl.pallas_call(
    _prop_body,
    grid=(2, _U // _RB),
    in_specs=[pl.BlockSpec((1, _RB, _H), lambda t, i: (t, i, 0)),
              pl.BlockSpec((1, _H, _D), lambda t, i: (t, 0, 0)),
              pl.BlockSpec((_RB, _D), lambda t, i: (t * (_U // _RB) + i, 0)),
              pl.BlockSpec((_RB, _D), lambda t, i: (t * (_U // _RB) + i, 0))],
    out_specs=[pl.BlockSpec((_RB, _D), lambda t, i: (t * (_U // _RB) + i, 0))] * 3,
    out_shape=[jax.ShapeDtypeStruct((_N, _D), jnp.float32)] * 3,
)


_SSLB = 512


def _loss_body(bpr_ref, e1_ref, e2_ref, e2b_ref, uh_ref, ih_ref, out_ref, acc):
    k = pl.program_id(0)
    j = pl.program_id(1)

    @pl.when((k == 0) & (j == 0))
    def _():
        ue = bpr_ref[0]
        pe = bpr_ref[1]
        ne = bpr_ref[2]
        x = jnp.sum(ue * ne, axis=1) - jnp.sum(ue * pe, axis=1)
        sp = jnp.maximum(x, 0.0) + jnp.log1p(jnp.exp(-jnp.abs(x)))
        acc[0] = jnp.mean(sp)
        ssq = (jnp.sum(bpr_ref[3] ** 2) + jnp.sum(bpr_ref[4] ** 2)
               + jnp.sum(bpr_ref[5] ** 2) + jnp.sum(uh_ref[...] ** 2)
               + jnp.sum(ih_ref[...] ** 2))
        acc[1] = _REG_L * 0.5 * ssq / _B
        acc[2] = 0.0

    v1 = e1_ref[0]
    v2 = e2_ref[0]
    v2b = e2b_ref[0]
    v1n = v1 / (jnp.sqrt(jnp.sum(v1 * v1, axis=1, keepdims=True)) + 1e-8)
    v2n = v2 / (jnp.sqrt(jnp.sum(v2 * v2, axis=1, keepdims=True)) + 1e-8)
    v2bn = v2b / (jnp.sqrt(jnp.sum(v2b * v2b, axis=1, keepdims=True)) + 1e-8)
    pos = jnp.sum(v1n * v2bn, axis=1) / _TEMP
    s_mat = lax.dot_general(v1n, v2n, (((1,), (1,)), ((), ())),
                            preferred_element_type=jnp.float32) / _TEMP
    ttl = jnp.log(jnp.sum(jnp.exp(s_mat), axis=1))
    acc[2] += jnp.sum(ttl - pos) / _B

    @pl.when((k == 3) & (j == (_B // _SSLB - 1)))
    def _():
        lane = lax.broadcasted_iota(jnp.int32, (8, 128), 1)
        out_ref[...] = jnp.where(
            lane == 0, acc[0],
            jnp.where(lane == 1, acc[1],
                      jnp.where(lane == 2, _SSL_L * acc[2], 0.0)))


_loss = pl.pallas_call(
    _loss_body,
    grid=(4, _B // _SSLB),
    in_specs=[pl.BlockSpec((6, _B, _D), lambda k, j: (0, 0, 0)),
              pl.BlockSpec((1, _SSLB, _D), lambda k, j: (6 + 2 * k, j, 0)),
              pl.BlockSpec((1, _B, _D), lambda k, j: (7 + 2 * k, 0, 0)),
              pl.BlockSpec((1, _SSLB, _D), lambda k, j: (7 + 2 * k, j, 0)),
              pl.BlockSpec((_D, _H), lambda k, j: (0, 0)),
              pl.BlockSpec((_D, _H), lambda k, j: (0, 0))],
    out_specs=pl.BlockSpec((8, 128), lambda k, j: (0, 0)),
    out_shape=jax.ShapeDtypeStruct((8, 128), jnp.float32),
    scratch_shapes=[pltpu.SMEM((4,), jnp.float32)],
)


def kernel(user, positive, negative, edge_index, graph_values,
           user_table, item_table, user_hyper, item_hyper):
    user = user.astype(jnp.int32)
    positive = positive.astype(jnp.int32)
    negative = negative.astype(jnp.int32)
    edge_index = edge_index.astype(jnp.int32)

    emb0 = jnp.concatenate([user_table, item_table], axis=0)
    proj = jnp.stack([user_hyper, item_hyper])

    pad = _EPAD - _E
    dst = jnp.concatenate([edge_index[0], jnp.zeros((pad,), jnp.int32)])
    src = jnp.concatenate([edge_index[1], jnp.zeros((pad,), jnp.int32)])
    val = jnp.concatenate([graph_values.astype(jnp.float32),
                           jnp.zeros((pad,), jnp.float32)])
    packed = jnp.stack([dst, src, lax.bitcast_convert_type(val, jnp.int32)])
    zeros_acc = jnp.zeros((_ACC_ROWS, _D), jnp.float32)

    hyp = _hyper(emb0, proj)

    emb = emb0
    accsum = emb0
    gs, hs = [], []
    for _l in range(2):
        lat = _latent(hyp, emb)
        g = _spmm()(packed, emb, zeros_acc)
        emb, h, accsum = _prop(hyp, lat, g, accsum)
        gs.append(g)
        hs.append(h)

    u = user
    p = positive + _U
    n = negative + _U
    idx_all = jnp.stack([u, p, n, u, p, n, u, u, p, p, u, u, p, p])
    gathered = _gather()(idx_all, accsum, emb0, gs[0], hs[0], gs[1], hs[1])

    out = _loss(gathered, gathered, gathered, gathered, user_hyper, item_hyper)
    return out[0, :3]


# speedup vs baseline: 1.0103x; 1.0103x over previous
import functools

import jax
import jax.numpy as jnp
from jax import lax
from jax.experimental import pallas as pl
from jax.experimental.pallas import tpu as pltpu
from jax.experimental.pallas import tpu_sc as plsc

_U = 25000
_N = 50000
_E = 800000
_D = 64
_H = 128
_TEMP = 0.2
_REG_L = 1e-4
_SSL_L = 0.1
_B = 4096

_NC = 2
_NS = 16
_CHUNK = 128
_CHT = 392
_EPT = _CHT * _CHUNK
_EPAD = _EPT * _NS
_DUMP = _U
_ACC_ROWS = 25088


def _spmm_body(packed_ref, vals_ref, emb_ref, zeros_ref, out_ref,
               edg0, edg1, val0, val1, dl0, dl1, rows0, rows1,
               acc_sh, smi0, smi1, smg0, smg1):
    c = lax.axis_index("c")
    s = lax.axis_index("s")
    lo = c * _U

    zb = s * (_ACC_ROWS // _NS)
    pltpu.sync_copy(zeros_ref.at[pl.ds(zb, _ACC_ROWS // _NS)],
                    acc_sh.at[pl.ds(zb, _ACC_ROWS // _NS)])
    plsc.subcore_barrier()

    ebase = s * _EPT
    maxoff = _EPAD - _CHUNK

    def _off(k):
        return jnp.minimum(ebase + k * _CHUNK, maxoff)

    def start_edges(k, edg, sem):
        pltpu.async_copy(packed_ref.at[:, pl.ds(_off(k), _CHUNK)], edg, sem)

    def start_vals(k, val, sem):
        pltpu.async_copy(vals_ref.at[pl.ds(_off(k), _CHUNK)], val, sem)

    def wait_idx(edg, val, sem):
        pltpu.make_async_copy(packed_ref.at[:, pl.ds(0, _CHUNK)], edg,
                              sem).wait()
        pltpu.make_async_copy(vals_ref.at[pl.ds(0, _CHUNK)], val, sem).wait()

    def start_gather(edg, rows, sem):
        pltpu.async_copy(emb_ref.at[edg.at[1]], rows, sem)

    def wait_gather(edg, rows, sem):
        pltpu.make_async_copy(emb_ref.at[edg.at[1]], rows, sem).wait()

    def compute_dl(edg, dl):
        for j in range(_CHUNK // 16):
            d16 = edg[0, pl.ds(j * 16, 16)]
            inr = (d16 >= lo) & (d16 < lo + _U)
            dl[pl.ds(j * 16, 16)] = jnp.where(inr, d16 - lo, _DUMP)

    def scale(val, rows):
        for j in range(_CHUNK // 16):
            v16 = val[pl.ds(j * 16, 16)]
            for e in range(16):
                vs = jnp.full((16,), v16[e])
                row = j * 16 + e
                for q in range(_D // 16):
                    rows[row, pl.ds(q * 16, 16)] = (
                        rows[row, pl.ds(q * 16, 16)] * vs)

    def process(k_next, edg, val, dl, rows, smi, smg):
        wait_gather(edg, rows, smg)
        compute_dl(edg, dl)
        start_edges(k_next, edg, smi)
        scale(val, rows)
        start_vals(k_next, val, smi)
        pltpu.sync_copy(rows, acc_sh.at[dl], add=True)
        wait_idx(edg, val, smi)
        start_gather(edg, rows, smg)

    start_edges(0, edg0, smi0)
    start_vals(0, val0, smi0)
    start_edges(1, edg1, smi1)
    start_vals(1, val1, smi1)
    wait_idx(edg0, val0, smi0)
    start_gather(edg0, rows0, smg0)
    wait_idx(edg1, val1, smi1)
    start_gather(edg1, rows1, smg1)

    def pair(i, carry):
        k = 2 * i
        process(k + 2, edg0, val0, dl0, rows0, smi0, smg0)
        process(k + 3, edg1, val1, dl1, rows1, smi1, smg1)
        return carry

    lax.fori_loop(0, _CHT // 2, pair, 0)

    wait_gather(edg0, rows0, smg0)
    wait_gather(edg1, rows1, smg1)
    plsc.subcore_barrier()

    @pl.when(s < 15)
    def _():
        base = s * 1568
        pltpu.sync_copy(acc_sh.at[pl.ds(base, 1568)],
                        out_ref.at[pl.ds(lo + base, 1568)])

    @pl.when(s == 15)
    def _():
        pltpu.sync_copy(acc_sh.at[pl.ds(15 * 1568, 1480)],
                        out_ref.at[pl.ds(lo + 15 * 1568, 1480)])


@functools.cache
def _spmm():
    return pl.kernel(
        _spmm_body,
        mesh=plsc.VectorSubcoreMesh(core_axis_name="c", subcore_axis_name="s", num_cores=_NC, num_subcores=_NS),
        out_type=jax.ShapeDtypeStruct((_N, _D), jnp.float32),
        compiler_params=pltpu.CompilerParams(use_tc_tiling_on_sc=False),
        scratch_types=[
            pltpu.VMEM((2, _CHUNK), jnp.int32),
            pltpu.VMEM((2, _CHUNK), jnp.int32),
            pltpu.VMEM((_CHUNK,), jnp.float32),
            pltpu.VMEM((_CHUNK,), jnp.float32),
            pltpu.VMEM((_CHUNK,), jnp.int32),
            pltpu.VMEM((_CHUNK,), jnp.int32),
            pltpu.VMEM((_CHUNK, _D), jnp.float32),
            pltpu.VMEM((_CHUNK, _D), jnp.float32),
            pltpu.VMEM_SHARED((_ACC_ROWS, _D), jnp.float32),
            pltpu.SemaphoreType.DMA,
            pltpu.SemaphoreType.DMA,
            pltpu.SemaphoreType.DMA,
            pltpu.SemaphoreType.DMA,
        ],
    )


_SMAP = (0, 0, 0, 1, 1, 1, 2, 3, 2, 3, 4, 5, 4, 5)


def _gather_body(idx_ref, s0, s1, s2, s3, s4, s5, out_ref, idx_v, rows_v, sem):
    w = lax.axis_index("s") * _NC + lax.axis_index("c")
    base = w * 128
    srcs = (s0, s1, s2, s3, s4, s5)
    for t in range(14):
        pltpu.sync_copy(idx_ref.at[t, pl.ds(base, 128)], idx_v)
        pltpu.async_copy(srcs[_SMAP[t]].at[idx_v], rows_v, sem).wait()
        pltpu.sync_copy(rows_v, out_ref.at[t, pl.ds(base, 128)])


@functools.cache
def _gather():
    return pl.kernel(
        _gather_body,
        mesh=plsc.VectorSubcoreMesh(core_axis_name="c", subcore_axis_name="s", num_cores=_NC, num_subcores=_NS),
        out_type=jax.ShapeDtypeStruct((14, _B, _D), jnp.float32),
        compiler_params=pltpu.CompilerParams(use_tc_tiling_on_sc=False),
        scratch_types=[
            pltpu.VMEM((128,), jnp.int32),
            pltpu.VMEM((128, _D), jnp.float32),
            pltpu.SemaphoreType.DMA,
        ],
    )


_RB = 1000


def _hyper_body(e_ref, p_ref, o_ref):
    o_ref[0] = jnp.dot(e_ref[...], p_ref[0], preferred_element_type=jnp.float32)


_hyper = pl.pallas_call(
    _hyper_body,
    grid=(2, _U // _RB),
    in_specs=[pl.BlockSpec((_RB, _D), lambda t, i: (t * (_U // _RB) + i, 0)),
              pl.BlockSpec((1, _D, _H), lambda t, i: (t, 0, 0))],
    out_specs=pl.BlockSpec((1, _RB, _H), lambda t, i: (t, i, 0)),
    out_shape=jax.ShapeDtypeStruct((2, _U, _H), jnp.float32),
)


def _latent_body(h_ref, e_ref, o_ref):
    i = pl.program_id(1)

    @pl.when(i == 0)
    def _():
        o_ref[...] = jnp.zeros_like(o_ref)

    a = h_ref[0]
    b = e_ref[...]
    o_ref[...] += lax.dot_general(a, b, (((0,), (0,)), ((), ())),
                                  preferred_element_type=jnp.float32)[None]


_latent = pl.pallas_call(
    _latent_body,
    grid=(2, _U // _RB),
    in_specs=[pl.BlockSpec((1, _RB, _H), lambda t, i: (t, i, 0)),
              pl.BlockSpec((_RB, _D), lambda t, i: (t * (_U // _RB) + i, 0))],
    out_specs=pl.BlockSpec((1, _H, _D), lambda t, i: (t, 0, 0)),
    out_shape=jax.ShapeDtypeStruct((2, _H, _D), jnp.float32),
)


def _prop_body(h_ref, l_ref, g_ref, a_ref, emb_o, h_o, acc_o):
    hblk = jnp.dot(h_ref[0], l_ref[0], preferred_element_type=jnp.float32)
    e = g_ref[...] + hblk
    emb_o[...] = e
    h_o[...] = hblk
    acc_o[...] = a_ref[...] + e


_prop = pl.pallas_call(
    _prop_body,
    grid=(2, _U // _RB),
    in_specs=[pl.BlockSpec((1, _RB, _H), lambda t, i: (t, i, 0)),
              pl.BlockSpec((1, _H, _D), lambda t, i: (t, 0, 0)),
              pl.BlockSpec((_RB, _D), lambda t, i: (t * (_U // _RB) + i, 0)),
              pl.BlockSpec((_RB, _D), lambda t, i: (t * (_U // _RB) + i, 0))],
    out_specs=[pl.BlockSpec((_RB, _D), lambda t, i: (t * (_U // _RB) + i, 0))] * 3,
    out_shape=[jax.ShapeDtypeStruct((_N, _D), jnp.float32)] * 3,
)


_SSLB = 512


def _loss_body(bpr_ref, e1_ref, e2_ref, e2b_ref, uh_ref, ih_ref, out_ref, acc):
    k = pl.program_id(0)
    j = pl.program_id(1)

    @pl.when((k == 0) & (j == 0))
    def _():
        ue = bpr_ref[0]
        pe = bpr_ref[1]
        ne = bpr_ref[2]
        x = jnp.sum(ue * ne, axis=1) - jnp.sum(ue * pe, axis=1)
        sp = jnp.maximum(x, 0.0) + jnp.log1p(jnp.exp(-jnp.abs(x)))
        acc[0] = jnp.mean(sp)
        ssq = (jnp.sum(bpr_ref[3] ** 2) + jnp.sum(bpr_ref[4] ** 2)
               + jnp.sum(bpr_ref[5] ** 2) + jnp.sum(uh_ref[...] ** 2)
               + jnp.sum(ih_ref[...] ** 2))
        acc[1] = _REG_L * 0.5 * ssq / _B
        acc[2] = 0.0

    v1 = e1_ref[0]
    v2 = e2_ref[0]
    v2b = e2b_ref[0]
    v1n = v1 / (jnp.sqrt(jnp.sum(v1 * v1, axis=1, keepdims=True)) + 1e-8)
    v2n = v2 / (jnp.sqrt(jnp.sum(v2 * v2, axis=1, keepdims=True)) + 1e-8)
    v2bn = v2b / (jnp.sqrt(jnp.sum(v2b * v2b, axis=1, keepdims=True)) + 1e-8)
    pos = jnp.sum(v1n * v2bn, axis=1) / _TEMP
    s_mat = lax.dot_general(v1n, v2n, (((1,), (1,)), ((), ())),
                            preferred_element_type=jnp.float32) / _TEMP
    ttl = jnp.log(jnp.sum(jnp.exp(s_mat), axis=1))
    acc[2] += jnp.sum(ttl - pos) / _B

    @pl.when((k == 3) & (j == (_B // _SSLB - 1)))
    def _():
        lane = lax.broadcasted_iota(jnp.int32, (8, 128), 1)
        out_ref[...] = jnp.where(
            lane == 0, acc[0],
            jnp.where(lane == 1, acc[1],
                      jnp.where(lane == 2, _SSL_L * acc[2], 0.0)))


_loss = pl.pallas_call(
    _loss_body,
    grid=(4, _B // _SSLB),
    in_specs=[pl.BlockSpec((6, _B, _D), lambda k, j: (0, 0, 0)),
              pl.BlockSpec((1, _SSLB, _D), lambda k, j: (6 + 2 * k, j, 0)),
              pl.BlockSpec((1, _B, _D), lambda k, j: (7 + 2 * k, 0, 0)),
              pl.BlockSpec((1, _SSLB, _D), lambda k, j: (7 + 2 * k, j, 0)),
              pl.BlockSpec((_D, _H), lambda k, j: (0, 0)),
              pl.BlockSpec((_D, _H), lambda k, j: (0, 0))],
    out_specs=pl.BlockSpec((8, 128), lambda k, j: (0, 0)),
    out_shape=jax.ShapeDtypeStruct((8, 128), jnp.float32),
    scratch_shapes=[pltpu.SMEM((4,), jnp.float32)],
)


def kernel(user, positive, negative, edge_index, graph_values,
           user_table, item_table, user_hyper, item_hyper):
    user = user.astype(jnp.int32)
    positive = positive.astype(jnp.int32)
    negative = negative.astype(jnp.int32)
    edge_index = edge_index.astype(jnp.int32)

    emb0 = jnp.concatenate([user_table, item_table], axis=0)
    proj = jnp.stack([user_hyper, item_hyper])

    pad = _EPAD - _E
    dst = jnp.concatenate([edge_index[0], jnp.zeros((pad,), jnp.int32)])
    src = jnp.concatenate([edge_index[1], jnp.zeros((pad,), jnp.int32)])
    val = jnp.concatenate([graph_values.astype(jnp.float32),
                           jnp.zeros((pad,), jnp.float32)])
    packed = jnp.stack([dst, src])
    zeros_acc = jnp.zeros((_ACC_ROWS, _D), jnp.float32)

    hyp = _hyper(emb0, proj)

    emb = emb0
    accsum = emb0
    gs, hs = [], []
    for _l in range(2):
        lat = _latent(hyp, emb)
        g = _spmm()(packed, val, emb, zeros_acc)
        emb, h, accsum = _prop(hyp, lat, g, accsum)
        gs.append(g)
        hs.append(h)

    u = user
    p = positive + _U
    n = negative + _U
    idx_all = jnp.stack([u, p, n, u, p, n, u, u, p, p, u, u, p, p])
    gathered = _gather()(idx_all, accsum, emb0, gs[0], hs[0], gs[1], hs[1])

    out = _loss(gathered, gathered, gathered, gathered, user_hyper, item_hyper)
    return out[0, :3]
